# Initial kernel scaffold; baseline (speedup 1.0000x reference)
import functools

import jax
import jax.numpy as jnp
from jax import lax
from jax.experimental import pallas as pl
from jax.experimental.pallas import tpu as pltpu
from jax.experimental.pallas import tpu_sc as plsc

VOCAB = 1000000
EMBED = 64
NUM_CLASS = 16
B = 16384

BLK = 2048
N_BLOCKS = B // BLK


def _sc_gather(table, idx2d):
    info = plsc.get_sparse_core_info()
    nc, ns = info.num_cores, info.num_subcores
    nw = nc * ns
    b_per_w = B // nw
    k = b_per_w // 128

    mesh = plsc.VectorSubcoreMesh(core_axis_name="c", subcore_axis_name="s")

    @functools.partial(
        pl.kernel,
        mesh=mesh,
        out_type=jax.ShapeDtypeStruct((B, EMBED), jnp.float32),
        scratch_types=[
            pltpu.VMEM((k, 128), jnp.int32),
            pltpu.VMEM((b_per_w, EMBED), jnp.float32),
            pltpu.SemaphoreType.DMA,
        ],
    )
    def gather_k(table_hbm, idx_hbm, out_hbm, idx_v, rows_v, sem):
        wid = lax.axis_index("s") * nc + lax.axis_index("c")
        pltpu.sync_copy(idx_hbm.at[pl.ds(wid * k, k)], idx_v)
        copies = [
            pltpu.async_copy(
                table_hbm.at[idx_v.at[j]],
                rows_v.at[pl.ds(j * 128, 128)],
                sem,
            )
            for j in range(k)
        ]
        for c in copies:
            c.wait()
        pltpu.sync_copy(rows_v, out_hbm.at[pl.ds(wid * b_per_w, b_per_w)])

    return gather_k(table, idx2d)


def _mlp_loss_body(emb_ref, lab_ref, w1_ref, b1_ref, w2_ref, b2_ref,
                   wfc_ref, bfc_ref, out_ref):
    x = emb_ref[...]
    h = jnp.maximum(
        jnp.dot(x, w1_ref[...], preferred_element_type=jnp.float32)
        + b1_ref[...], 0.0)
    h = jnp.maximum(
        jnp.dot(h, w2_ref[...], preferred_element_type=jnp.float32)
        + b2_ref[...], 0.0)
    logits = (jnp.dot(h, wfc_ref[...], preferred_element_type=jnp.float32)
              + bfc_ref[...])
    m = jnp.max(logits, axis=1, keepdims=True)
    lse = jnp.log(jnp.sum(jnp.exp(logits - m), axis=1, keepdims=True)) + m
    cls = lax.broadcasted_iota(jnp.int32, (BLK, NUM_CLASS), 1)
    picked = jnp.sum(jnp.where(cls == lab_ref[...], logits, 0.0),
                     axis=1, keepdims=True)
    part = jnp.sum(lse - picked)

    @pl.when(pl.program_id(0) == 0)
    def _init():
        out_ref[0, 0] = 0.0

    out_ref[0, 0] += part


def _mlp_loss(emb, labels2d, W1, b1, W2, b2, Wfc, bfc):
    return pl.pallas_call(
        _mlp_loss_body,
        grid=(N_BLOCKS,),
        in_specs=[
            pl.BlockSpec((BLK, EMBED), lambda i: (i, 0)),
            pl.BlockSpec((BLK, 1), lambda i: (i, 0)),
            pl.BlockSpec((EMBED, EMBED), lambda i: (0, 0)),
            pl.BlockSpec((1, EMBED), lambda i: (0, 0)),
            pl.BlockSpec((EMBED, EMBED), lambda i: (0, 0)),
            pl.BlockSpec((1, EMBED), lambda i: (0, 0)),
            pl.BlockSpec((EMBED, NUM_CLASS), lambda i: (0, 0)),
            pl.BlockSpec((1, NUM_CLASS), lambda i: (0, 0)),
        ],
        out_specs=pl.BlockSpec((1, 1), lambda i: (0, 0)),
        out_shape=jax.ShapeDtypeStruct((1, 1), jnp.float32),
    )(emb, labels2d, W1, b1, W2, b2, Wfc, bfc)


def kernel(text, offsets, labels, table, W1, b1, W2, b2, Wfc, bfc):
    idx2d = text.astype(jnp.int32).reshape(B // 128, 128)
    emb = _sc_gather(table, idx2d)
    loss_sum = _mlp_loss(
        emb,
        labels.astype(jnp.int32).reshape(B, 1),
        W1, b1.reshape(1, EMBED),
        W2, b2.reshape(1, EMBED),
        Wfc, bfc.reshape(1, NUM_CLASS),
    )
    return loss_sum[0, 0] / B


# speedup vs baseline: 1.2282x; 1.2282x over previous
import functools

import jax
import jax.numpy as jnp
from jax import lax
from jax.experimental import pallas as pl
from jax.experimental.pallas import tpu as pltpu
from jax.experimental.pallas import tpu_sc as plsc

VOCAB = 1000000
EMBED = 64
NUM_CLASS = 16
B = 16384

BLK = 2048
N_BLOCKS = B // BLK


def _sc_gather(table, idx2d):
    info = plsc.get_sparse_core_info()
    nc, ns = info.num_cores, info.num_subcores
    nw = nc * ns
    b_per_w = B // nw
    k = b_per_w // 128

    mesh = plsc.VectorSubcoreMesh(core_axis_name="c", subcore_axis_name="s")

    @functools.partial(
        pl.kernel,
        mesh=mesh,
        out_type=jax.ShapeDtypeStruct((B, EMBED), jnp.float32),
        compiler_params=pltpu.CompilerParams(use_tc_tiling_on_sc=False),
        scratch_types=[
            pltpu.VMEM((k, 128), jnp.int32),
            pltpu.VMEM((b_per_w, EMBED), jnp.float32),
            pltpu.SemaphoreType.DMA,
        ],
    )
    def gather_k(table_hbm, idx_hbm, out_hbm, idx_v, rows_v, sem):
        wid = lax.axis_index("s") * nc + lax.axis_index("c")
        pltpu.sync_copy(idx_hbm.at[pl.ds(wid * k, k)], idx_v)
        copies = [
            pltpu.async_copy(
                table_hbm.at[idx_v.at[j]],
                rows_v.at[pl.ds(j * 128, 128)],
                sem,
            )
            for j in range(k)
        ]
        for c in copies:
            c.wait()
        pltpu.sync_copy(rows_v, out_hbm.at[pl.ds(wid * b_per_w, b_per_w)])

    return gather_k(table, idx2d)


def _mlp_loss_body(emb_ref, lab_ref, w1_ref, b1_ref, w2_ref, b2_ref,
                   wfc_ref, bfc_ref, out_ref):
    x = emb_ref[...]
    h = jnp.maximum(
        jnp.dot(x, w1_ref[...], preferred_element_type=jnp.float32)
        + b1_ref[...], 0.0)
    h = jnp.maximum(
        jnp.dot(h, w2_ref[...], preferred_element_type=jnp.float32)
        + b2_ref[...], 0.0)
    logits = (jnp.dot(h, wfc_ref[...], preferred_element_type=jnp.float32)
              + bfc_ref[...])
    m = jnp.max(logits, axis=1, keepdims=True)
    lse = jnp.log(jnp.sum(jnp.exp(logits - m), axis=1, keepdims=True)) + m
    cls = lax.broadcasted_iota(jnp.int32, (BLK, NUM_CLASS), 1)
    picked = jnp.sum(jnp.where(cls == lab_ref[...], logits, 0.0),
                     axis=1, keepdims=True)
    part = jnp.sum(lse - picked, axis=(0, 1), keepdims=True)

    @pl.when(pl.program_id(0) == 0)
    def _init():
        out_ref[...] = jnp.zeros_like(out_ref)

    out_ref[...] += part


def _mlp_loss(emb, labels2d, W1, b1, W2, b2, Wfc, bfc):
    return pl.pallas_call(
        _mlp_loss_body,
        grid=(N_BLOCKS,),
        in_specs=[
            pl.BlockSpec((BLK, EMBED), lambda i: (i, 0)),
            pl.BlockSpec((BLK, 1), lambda i: (i, 0)),
            pl.BlockSpec((EMBED, EMBED), lambda i: (0, 0)),
            pl.BlockSpec((1, EMBED), lambda i: (0, 0)),
            pl.BlockSpec((EMBED, EMBED), lambda i: (0, 0)),
            pl.BlockSpec((1, EMBED), lambda i: (0, 0)),
            pl.BlockSpec((EMBED, NUM_CLASS), lambda i: (0, 0)),
            pl.BlockSpec((1, NUM_CLASS), lambda i: (0, 0)),
        ],
        out_specs=pl.BlockSpec((1, 1), lambda i: (0, 0)),
        out_shape=jax.ShapeDtypeStruct((1, 1), jnp.float32),
    )(emb, labels2d, W1, b1, W2, b2, Wfc, bfc)


def kernel(text, offsets, labels, table, W1, b1, W2, b2, Wfc, bfc):
    idx2d = text.astype(jnp.int32).reshape(B // 128, 128)
    emb = _sc_gather(table, idx2d)
    loss_sum = _mlp_loss(
        emb,
        labels.astype(jnp.int32).reshape(B, 1),
        W1, b1.reshape(1, EMBED),
        W2, b2.reshape(1, EMBED),
        Wfc, bfc.reshape(1, NUM_CLASS),
    )
    return loss_sum[0, 0] / B


# speedup vs baseline: 3.5948x; 2.9268x over previous
import functools

import jax
import jax.numpy as jnp
from jax import lax
from jax.experimental import pallas as pl
from jax.experimental.pallas import tpu as pltpu
from jax.experimental.pallas import tpu_sc as plsc

VOCAB = 1000000
EMBED = 64
NUM_CLASS = 16
B = 16384

BLK = 2048
N_BLOCKS = B // BLK

NBUF = 8


def _sc_gather_cols(tableT, text):
    info = plsc.get_sparse_core_info()
    nc, ns = info.num_cores, info.num_subcores
    nw = nc * ns
    b_per_w = B // nw
    n_outer = b_per_w // NBUF

    mesh = plsc.VectorSubcoreMesh(core_axis_name="c", subcore_axis_name="s")

    @functools.partial(
        pl.kernel,
        mesh=mesh,
        out_type=jax.ShapeDtypeStruct((EMBED, B), jnp.float32),
        compiler_params=pltpu.CompilerParams(needs_layout_passes=False),
        scratch_types=[
            pltpu.VMEM((NBUF, EMBED, 128), jnp.float32),
            pltpu.VMEM((EMBED, b_per_w), jnp.float32),
            pltpu.VMEM((b_per_w,), jnp.int32),
        ] + [pltpu.SemaphoreType.DMA] * NBUF,
    )
    def gather_k(tableT_hbm, text_hbm, out_hbm, tiles_v, stage_v, idx_v,
                 *fsems):
        wid = lax.axis_index("s") * nc + lax.axis_index("c")
        base = wid * b_per_w

        pltpu.sync_copy(text_hbm.at[pl.ds(base, b_per_w)], idx_v)

        def get_idx(gbase, c):
            vec = idx_v[pl.ds(gbase, 16)]
            sel = jnp.where(lax.iota(jnp.int32, 16) == c, vec, 0)
            return jnp.sum(sel)

        def fetch(i, slot):
            off = pl.multiple_of((i >> 7) << 7, 128)
            pltpu.async_copy(
                tableT_hbm.at[:, pl.ds(off, 128)],
                tiles_v.at[slot],
                fsems[slot],
            )

        for s in range(NBUF):
            fetch(get_idx(0, s), s)

        def outer(o, carry):
            gbase = (o >> 1) * 16
            half = (o & 1) * NBUF
            for s in range(NBUF):
                pltpu.make_async_copy(
                    tableT_hbm.at[:, pl.ds(0, 128)],
                    tiles_v.at[s],
                    fsems[s],
                ).wait()
                i = get_idx(gbase, half + s)
                lane = jnp.full((16,), i & 127, jnp.int32)
                col = jnp.full((16,), o * NBUF + s, jnp.int32)
                for k in range(EMBED // 16):
                    rows = lax.iota(jnp.int32, 16) + (16 * k)
                    vals = plsc.load_gather(tiles_v.at[s], [rows, lane])
                    plsc.store_scatter(stage_v, [rows, col], vals)
                @pl.when(o < n_outer - 1)
                def _refill():
                    fetch(get_idx(((o + 1) >> 1) * 16, ((o + 1) & 1) * NBUF + s),
                          s)

            return carry

        lax.fori_loop(0, n_outer, outer, 0, unroll=False)

        pltpu.sync_copy(stage_v, out_hbm.at[:, pl.ds(base, b_per_w)])

    return gather_k(tableT, text)


def _mlp_loss_body(emb_ref, lab_ref, w1_ref, b1_ref, w2_ref, b2_ref,
                   wfc_ref, bfc_ref, out_ref):
    x = emb_ref[...]
    cdims = (((0,), (0,)), ((), ()))
    h = jnp.maximum(
        lax.dot_general(w1_ref[...], x, cdims,
                        preferred_element_type=jnp.float32) + b1_ref[...], 0.0)
    h = jnp.maximum(
        lax.dot_general(w2_ref[...], h, cdims,
                        preferred_element_type=jnp.float32) + b2_ref[...], 0.0)
    logits = lax.dot_general(wfc_ref[...], h, cdims,
                             preferred_element_type=jnp.float32) + bfc_ref[...]
    m = jnp.max(logits, axis=0, keepdims=True)
    lse = jnp.log(jnp.sum(jnp.exp(logits - m), axis=0, keepdims=True)) + m
    lab = lab_ref[...].reshape(1, BLK)
    cls = lax.broadcasted_iota(jnp.int32, (NUM_CLASS, BLK), 0)
    picked = jnp.sum(jnp.where(cls == lab, logits, 0.0),
                     axis=0, keepdims=True)
    part = jnp.sum(lse - picked, axis=(0, 1), keepdims=True)

    @pl.when(pl.program_id(0) == 0)
    def _init():
        out_ref[...] = jnp.zeros_like(out_ref)

    out_ref[...] += part


def _mlp_loss(embT, labels3d, W1, b1, W2, b2, Wfc, bfc):
    return pl.pallas_call(
        _mlp_loss_body,
        grid=(N_BLOCKS,),
        in_specs=[
            pl.BlockSpec((EMBED, BLK), lambda i: (0, i)),
            pl.BlockSpec((1, 1, BLK), lambda i: (i, 0, 0)),
            pl.BlockSpec((EMBED, EMBED), lambda i: (0, 0)),
            pl.BlockSpec((EMBED, 1), lambda i: (0, 0)),
            pl.BlockSpec((EMBED, EMBED), lambda i: (0, 0)),
            pl.BlockSpec((EMBED, 1), lambda i: (0, 0)),
            pl.BlockSpec((EMBED, NUM_CLASS), lambda i: (0, 0)),
            pl.BlockSpec((NUM_CLASS, 1), lambda i: (0, 0)),
        ],
        out_specs=pl.BlockSpec((1, 1), lambda i: (0, 0)),
        out_shape=jax.ShapeDtypeStruct((1, 1), jnp.float32),
    )(embT, labels3d, W1, b1, W2, b2, Wfc, bfc)


def kernel(text, offsets, labels, table, W1, b1, W2, b2, Wfc, bfc):
    embT = _sc_gather_cols(table.T, text.astype(jnp.int32))
    loss_sum = _mlp_loss(
        embT,
        labels.astype(jnp.int32).reshape(N_BLOCKS, 1, BLK),
        W1, b1.reshape(EMBED, 1),
        W2, b2.reshape(EMBED, 1),
        Wfc, bfc.reshape(NUM_CLASS, 1),
    )
    return loss_sum[0, 0] / B
